# baseline (speedup 1.0000x reference)
import functools

import jax
import jax.numpy as jnp
from jax import lax
from jax.experimental import pallas as pl
from jax.experimental.pallas import tpu as pltpu
from jax.experimental.pallas import tpu_sc as plsc

_NUM_CORES = 2
_NUM_SUBCORES = 16
_NUM_WORKERS = _NUM_CORES * _NUM_SUBCORES


def kernel(team_ID, table):
    (batch,) = team_ID.shape
    vocab, dim = table.shape
    assert batch % (_NUM_WORKERS * 16) == 0
    b_per_w = batch // _NUM_WORKERS

    idx = team_ID.astype(jnp.int32).reshape(1, batch)

    mesh = plsc.VectorSubcoreMesh(core_axis_name="c", subcore_axis_name="s")

    @functools.partial(
        pl.kernel,
        mesh=mesh,
        out_type=jax.ShapeDtypeStruct((batch, dim), table.dtype),
        compiler_params=pltpu.CompilerParams(disable_bounds_checks=True),
        scratch_types=[
            pltpu.VMEM((1, b_per_w), jnp.int32),
            pltpu.SemaphoreType.DMA,
        ],
    )
    def gather_kernel(idx_hbm, table_hbm, out_hbm, idx_v, sem):
        wid = lax.axis_index("s") * _NUM_CORES + lax.axis_index("c")
        base = wid * b_per_w
        pltpu.sync_copy(idx_hbm.at[:, pl.ds(base, b_per_w)], idx_v)

        @plsc.parallel_loop(0, b_per_w, step=16, unroll=2)
        def _(i):
            v = idx_v[0, pl.ds(i, 16)]
            for j in range(16):
                pltpu.async_copy(
                    table_hbm.at[pl.ds(v[j], 1)],
                    out_hbm.at[pl.ds(base + i + j, 1)],
                    sem,
                )

        pltpu.make_async_copy(
            table_hbm.at[pl.ds(0, b_per_w)],
            out_hbm.at[pl.ds(base, b_per_w)],
            sem,
        ).wait()

    return gather_kernel(idx, table)


# speedup vs baseline: 1.7179x; 1.7179x over previous
import functools

import jax
import jax.numpy as jnp
from jax import lax
from jax.experimental import pallas as pl
from jax.experimental.pallas import tpu as pltpu
from jax.experimental.pallas import tpu_sc as plsc

_NUM_CORES = 2
_NUM_SUBCORES = 16
_NUM_WORKERS = _NUM_CORES * _NUM_SUBCORES


def kernel(team_ID, table):
    (batch,) = team_ID.shape
    vocab, dim = table.shape
    assert batch % (_NUM_WORKERS * 16) == 0
    b_per_w = batch // _NUM_WORKERS
    half = b_per_w // 2

    idx = team_ID.astype(jnp.int32).reshape(1, batch)

    mesh = plsc.VectorSubcoreMesh(core_axis_name="c", subcore_axis_name="s")

    @functools.partial(
        pl.kernel,
        mesh=mesh,
        out_type=jax.ShapeDtypeStruct((batch, dim), table.dtype),
        compiler_params=pltpu.CompilerParams(disable_bounds_checks=True),
        scratch_types=[
            pltpu.VMEM((1, b_per_w), jnp.int32),
            pltpu.VMEM((half, dim), jnp.float32),
            pltpu.VMEM_SHARED((_NUM_SUBCORES, half, dim), jnp.float32),
            pltpu.SemaphoreType.DMA,
            pltpu.SemaphoreType.DMA,
        ],
    )
    def gather_kernel(idx_hbm, table_hbm, out_hbm, idx_v, rows_v, rows_sh,
                      sem_a, sem_b):
        wid = lax.axis_index("s") * _NUM_CORES + lax.axis_index("c")
        sid = lax.axis_index("s")
        base = wid * b_per_w
        pltpu.sync_copy(idx_hbm.at[:, pl.ds(base, b_per_w)], idx_v)

        @plsc.parallel_loop(0, half, step=16)
        def _(i):
            v = idx_v[0, pl.ds(i, 16)]
            for j in range(16):
                pltpu.async_copy(
                    table_hbm.at[pl.ds(v[j], 1)],
                    rows_v.at[pl.ds(i + j, 1)],
                    sem_a,
                )

        @plsc.parallel_loop(half, b_per_w, step=16)
        def _(i):
            v = idx_v[0, pl.ds(i, 16)]
            for j in range(16):
                pltpu.async_copy(
                    table_hbm.at[pl.ds(v[j], 1)],
                    rows_sh.at[sid, pl.ds(i - half + j, 1)],
                    sem_b,
                )

        pltpu.make_async_copy(
            table_hbm.at[pl.ds(0, half)], rows_v, sem_a
        ).wait()
        pltpu.sync_copy(rows_v, out_hbm.at[pl.ds(base, half)])
        pltpu.make_async_copy(
            table_hbm.at[pl.ds(0, half)], rows_sh.at[sid], sem_b
        ).wait()
        pltpu.sync_copy(rows_sh.at[sid], out_hbm.at[pl.ds(base + half, half)])

    return gather_kernel(idx, table)
